# baseline (speedup 1.0000x reference)
import functools

import jax
import jax.numpy as jnp
from jax import lax
from jax.experimental import pallas as pl
from jax.experimental.pallas import tpu as pltpu
from jax.experimental.pallas import tpu_sc as plsc

N = 10000
E = 320000
NC = 2
NS = 16
NW = NC * NS
EPW = E // NW
K = 128
NCHUNK = 80
KCAP = NCHUNK * K
PADROW = N
NP = 10240
ZR = NP // NS
NBUF = 8
GA = 4
DEG_Q = 8

_MESH = dict(core_axis_name="c", subcore_axis_name="s")
_SC_PARAMS = pltpu.CompilerParams(use_tc_tiling_on_sc=False)


def _stage_indices(ei_hbm, w, sidx_v, didx_v):
    base = w * EPW
    pltpu.sync_copy(ei_hbm.at[0, pl.ds(base, EPW)], sidx_v.at[pl.ds(0, EPW)])
    pltpu.sync_copy(ei_hbm.at[1, pl.ds(base, EPW)], didx_v.at[pl.ds(0, EPW)])
    zid = jnp.zeros((16,), jnp.int32)
    pad = jnp.full((16,), PADROW, jnp.int32)
    for j in range((KCAP - EPW) // 16):
        sidx_v[pl.ds(EPW + j * 16, 16)] = zid
        didx_v[pl.ds(EPW + j * 16, 16)] = pad


def _sc_degree(ei, ones_k, zeros_row):
    @functools.partial(
        pl.kernel,
        out_type=jax.ShapeDtypeStruct((NC, NP, 16), jnp.float32),
        mesh=plsc.VectorSubcoreMesh(**_MESH),
        scratch_types=[
            pltpu.VMEM((KCAP,), jnp.int32),
            pltpu.VMEM((KCAP,), jnp.int32),
            pltpu.VMEM((K, 16), jnp.float32),
            pltpu.VMEM_SHARED((NP, 16), jnp.float32),
            pltpu.SemaphoreType.DMA,
        ],
        compiler_params=_SC_PARAMS,
    )
    def deg_kernel(ei_hbm, ones_hbm, zeros_hbm, out_hbm, sidx_v, didx_v,
                   ones_v, acc_sh, sem):
        c = lax.axis_index("c")
        s = lax.axis_index("s")
        w = c * NS + s
        _stage_indices(ei_hbm, w, sidx_v, didx_v)
        pltpu.sync_copy(ones_hbm, ones_v)
        pltpu.sync_copy(zeros_hbm, acc_sh.at[pl.ds(s * ZR, ZR)])
        plsc.subcore_barrier()

        def group(g, carry):
            for b in range(DEG_Q):
                pltpu.async_copy(
                    ones_v,
                    acc_sh.at[didx_v.at[pl.ds((g * DEG_Q + b) * K, K)]],
                    sem, add=True)
            for b in range(DEG_Q):
                pltpu.make_async_copy(
                    ones_v,
                    acc_sh.at[didx_v.at[pl.ds((g * DEG_Q + b) * K, K)]],
                    sem).wait()
            return carry

        lax.fori_loop(0, NCHUNK // DEG_Q, group, 0)
        plsc.subcore_barrier()
        pltpu.sync_copy(acc_sh.at[pl.ds(s * ZR, ZR)],
                        out_hbm.at[c, pl.ds(s * ZR, ZR)])

    return deg_kernel(ei, ones_k, zeros_row)


def _sc_aggregate(table, ei, zeros_row, width):
    @functools.partial(
        pl.kernel,
        out_type=jax.ShapeDtypeStruct((NC, NP, width), jnp.float32),
        mesh=plsc.VectorSubcoreMesh(**_MESH),
        scratch_types=[
            pltpu.VMEM((KCAP,), jnp.int32),
            pltpu.VMEM((KCAP,), jnp.int32),
            [pltpu.VMEM((K, width), jnp.float32)] * NBUF,
            pltpu.VMEM_SHARED((NP, width), jnp.float32),
            [pltpu.SemaphoreType.DMA] * NBUF,
            [pltpu.SemaphoreType.DMA] * NBUF,
        ],
        compiler_params=_SC_PARAMS,
    )
    def agg_kernel(table_hbm, ei_hbm, zeros_hbm, out_hbm, sidx_v, didx_v,
                   rows, acc_sh, gsems, ssems):
        c = lax.axis_index("c")
        s = lax.axis_index("s")
        w = c * NS + s
        _stage_indices(ei_hbm, w, sidx_v, didx_v)
        pltpu.sync_copy(zeros_hbm, acc_sh.at[pl.ds(s * ZR, ZR)])
        plsc.subcore_barrier()

        def gather(chunk, b):
            pltpu.async_copy(table_hbm.at[sidx_v.at[pl.ds(chunk * K, K)]],
                             rows[b], gsems[b])

        def gather_wait(chunk, b):
            pltpu.make_async_copy(
                table_hbm.at[sidx_v.at[pl.ds(chunk * K, K)]], rows[b],
                gsems[b]).wait()

        def scatter(chunk, b):
            pltpu.async_copy(rows[b],
                             acc_sh.at[didx_v.at[pl.ds(chunk * K, K)]],
                             ssems[b], add=True)

        def scatter_wait(b):
            pltpu.make_async_copy(rows[b],
                                  acc_sh.at[didx_v.at[pl.ds(0, K)]],
                                  ssems[b]).wait()

        for b in range(GA):
            gather(b, b)

        def group(g, carry):
            for u in range(NBUF):
                chunk = g * NBUF + u
                gather_wait(chunk, u)
                scatter(chunk, u)
                bf = (u + GA) % NBUF

                @pl.when(chunk + GA < NCHUNK)
                def _():
                    @pl.when(chunk >= NBUF - GA)
                    def _():
                        scatter_wait(bf)

                    gather(chunk + GA, bf)

            return carry

        lax.fori_loop(0, NCHUNK // NBUF, group, 0)
        for b in range(NBUF):
            scatter_wait(b)
        plsc.subcore_barrier()
        pltpu.sync_copy(acc_sh.at[pl.ds(s * ZR, ZR)],
                        out_hbm.at[c, pl.ds(s * ZR, ZR)])

    return agg_kernel(table, ei, zeros_row)


_BM = 1000


def _tc_matmul1(x, w1p):
    def body(x_ref, w_ref, h_ref):
        h_ref[...] = jnp.dot(x_ref[...], w_ref[...],
                             preferred_element_type=jnp.float32)

    return pl.pallas_call(
        body,
        grid=(N // _BM,),
        in_specs=[
            pl.BlockSpec((_BM, 128), lambda i: (i, 0)),
            pl.BlockSpec((128, 64), lambda i: (0, 0)),
        ],
        out_specs=pl.BlockSpec((_BM, 64), lambda i: (i, 0)),
        out_shape=jax.ShapeDtypeStruct((N, 64), jnp.float32),
    )(x, w1p)


def _tc_mid1(h1, degp):
    def body(h_ref, deg_ref, hs_ref, dinv_ref):
        deg = 1.0 + deg_ref[0] + deg_ref[1]
        dinv = lax.rsqrt(deg)
        hs_ref[...] = h_ref[...] * dinv[:, 0:1]
        dinv_ref[...] = dinv

    return pl.pallas_call(
        body,
        grid=(N // _BM,),
        in_specs=[
            pl.BlockSpec((_BM, 64), lambda i: (i, 0)),
            pl.BlockSpec((NC, _BM, 16), lambda i: (0, i, 0)),
        ],
        out_specs=[
            pl.BlockSpec((_BM, 64), lambda i: (i, 0)),
            pl.BlockSpec((_BM, 16), lambda i: (i, 0)),
        ],
        out_shape=[
            jax.ShapeDtypeStruct((N, 64), jnp.float32),
            jax.ShapeDtypeStruct((N, 16), jnp.float32),
        ],
    )(h1, degp)


def _tc_mid2(p, hs1, dinv, b1p, w2p):
    def body(p_ref, hs_ref, dinv_ref, b1_ref, w2_ref, out_ref):
        d1 = dinv_ref[...][:, 0:1]
        a = (p_ref[0] + p_ref[1] + hs_ref[...]) * d1 + b1_ref[...]
        r = jnp.maximum(a, 0.0)
        h2 = jnp.dot(r, w2_ref[...], preferred_element_type=jnp.float32)
        out_ref[...] = h2 * d1

    return pl.pallas_call(
        body,
        grid=(N // _BM,),
        in_specs=[
            pl.BlockSpec((NC, _BM, 64), lambda i: (0, i, 0)),
            pl.BlockSpec((_BM, 64), lambda i: (i, 0)),
            pl.BlockSpec((_BM, 16), lambda i: (i, 0)),
            pl.BlockSpec((1, 64), lambda i: (0, 0)),
            pl.BlockSpec((64, 16), lambda i: (0, 0)),
        ],
        out_specs=pl.BlockSpec((_BM, 16), lambda i: (i, 0)),
        out_shape=jax.ShapeDtypeStruct((N, 16), jnp.float32),
    )(p, hs1, dinv, b1p, w2p)


def _tc_final(q, hs2, dinv, b2p):
    def body(q_ref, hs_ref, dinv_ref, b2_ref, out_ref):
        d1 = dinv_ref[...][:, 0:1]
        z = (q_ref[0] + q_ref[1] + hs_ref[...]) * d1
        logits = z[:, 0:2] + b2_ref[...]
        m = jnp.max(logits, axis=1, keepdims=True)
        e = jnp.exp(logits - m)
        lse = m + jnp.log(e[:, 0:1] + e[:, 1:2])
        out_ref[...] = logits - lse

    return pl.pallas_call(
        body,
        grid=(N // _BM,),
        in_specs=[
            pl.BlockSpec((NC, _BM, 16), lambda i: (0, i, 0)),
            pl.BlockSpec((_BM, 16), lambda i: (i, 0)),
            pl.BlockSpec((_BM, 16), lambda i: (i, 0)),
            pl.BlockSpec((1, 2), lambda i: (0, 0)),
        ],
        out_specs=pl.BlockSpec((_BM, 2), lambda i: (i, 0)),
        out_shape=jax.ShapeDtypeStruct((N, 2), jnp.float32),
    )(q, hs2, dinv, b2p)


def kernel(x, edge_index, W1, b1, W2, b2):
    f_in, h_dim = W1.shape
    c_dim = W2.shape[1]
    w1p = jnp.zeros((f_in, 64), jnp.float32).at[:, :h_dim].set(W1)
    w2p = jnp.zeros((64, 16), jnp.float32).at[:h_dim, :c_dim].set(W2)
    b1p = jnp.zeros((1, 64), jnp.float32).at[0, :h_dim].set(b1)
    b2p = b2.reshape(1, c_dim)
    ones_k = jnp.ones((K, 16), jnp.float32)
    zeros16 = jnp.zeros((ZR, 16), jnp.float32)
    zeros64 = jnp.zeros((ZR, 64), jnp.float32)

    h1 = _tc_matmul1(x, w1p)
    degp = _sc_degree(edge_index, ones_k, zeros16)
    hs1, dinv = _tc_mid1(h1, degp)
    p = _sc_aggregate(hs1, edge_index, zeros64, 64)
    hs2 = _tc_mid2(p, hs1, dinv, b1p, w2p)
    q = _sc_aggregate(hs2, edge_index, zeros16, 16)
    return _tc_final(q, hs2, dinv, b2p)


# speedup vs baseline: 1.6910x; 1.6910x over previous
import functools

import jax
import jax.numpy as jnp
from jax import lax
from jax.experimental import pallas as pl
from jax.experimental.pallas import tpu as pltpu
from jax.experimental.pallas import tpu_sc as plsc

N = 10000
E = 320000
NC = 2
NS = 16
NW = NC * NS
EPW = E // NW
K = 128
NCHUNK = 80
KCAP = NCHUNK * K
PADROW = N
NP = 10240
ZR = NP // NS
NBUF = 2
GA = 1
DEG_Q = 8

_MESH = dict(core_axis_name="c", subcore_axis_name="s")
_SC_PARAMS = pltpu.CompilerParams(use_tc_tiling_on_sc=False)


def _stage_indices(ei_hbm, w, sidx_v, didx_v):
    base = w * EPW
    pltpu.sync_copy(ei_hbm.at[0, pl.ds(base, EPW)], sidx_v.at[pl.ds(0, EPW)])
    pltpu.sync_copy(ei_hbm.at[1, pl.ds(base, EPW)], didx_v.at[pl.ds(0, EPW)])
    zid = jnp.zeros((16,), jnp.int32)
    pad = jnp.full((16,), PADROW, jnp.int32)
    for j in range((KCAP - EPW) // 16):
        sidx_v[pl.ds(EPW + j * 16, 16)] = zid
        didx_v[pl.ds(EPW + j * 16, 16)] = pad


def _sc_degree(ei, ones_k, zeros_row):
    @functools.partial(
        pl.kernel,
        out_type=jax.ShapeDtypeStruct((NC, NP, 16), jnp.float32),
        mesh=plsc.VectorSubcoreMesh(**_MESH),
        scratch_types=[
            pltpu.VMEM((KCAP,), jnp.int32),
            pltpu.VMEM((KCAP,), jnp.int32),
            pltpu.VMEM((K, 16), jnp.float32),
            pltpu.VMEM_SHARED((NP, 16), jnp.float32),
            pltpu.SemaphoreType.DMA,
        ],
        compiler_params=_SC_PARAMS,
    )
    def deg_kernel(ei_hbm, ones_hbm, zeros_hbm, out_hbm, sidx_v, didx_v,
                   ones_v, acc_sh, sem):
        c = lax.axis_index("c")
        s = lax.axis_index("s")
        w = c * NS + s
        _stage_indices(ei_hbm, w, sidx_v, didx_v)
        pltpu.sync_copy(ones_hbm, ones_v)
        pltpu.sync_copy(zeros_hbm, acc_sh.at[pl.ds(s * ZR, ZR)])
        plsc.subcore_barrier()

        def group(g, carry):
            for b in range(DEG_Q):
                pltpu.async_copy(
                    ones_v,
                    acc_sh.at[didx_v.at[pl.ds((g * DEG_Q + b) * K, K)]],
                    sem, add=True)
            for b in range(DEG_Q):
                pltpu.make_async_copy(
                    ones_v,
                    acc_sh.at[didx_v.at[pl.ds((g * DEG_Q + b) * K, K)]],
                    sem).wait()
            return carry

        lax.fori_loop(0, NCHUNK // DEG_Q, group, 0)
        plsc.subcore_barrier()
        pltpu.sync_copy(acc_sh.at[pl.ds(s * ZR, ZR)],
                        out_hbm.at[c, pl.ds(s * ZR, ZR)])

    return deg_kernel(ei, ones_k, zeros_row)


def _sc_aggregate(table, ei, zeros_row, width):
    @functools.partial(
        pl.kernel,
        out_type=jax.ShapeDtypeStruct((NC, NP, width), jnp.float32),
        mesh=plsc.VectorSubcoreMesh(**_MESH),
        scratch_types=[
            pltpu.VMEM((KCAP,), jnp.int32),
            pltpu.VMEM((KCAP,), jnp.int32),
            [pltpu.VMEM((K, width), jnp.float32)] * NBUF,
            pltpu.VMEM_SHARED((N, width), jnp.float32),
            pltpu.VMEM_SHARED((NP, width), jnp.float32),
            [pltpu.SemaphoreType.DMA] * NBUF,
            [pltpu.SemaphoreType.DMA] * NBUF,
        ],
        compiler_params=_SC_PARAMS,
    )
    def agg_kernel(table_hbm, ei_hbm, zeros_hbm, out_hbm, sidx_v, didx_v,
                   rows, table_sh, acc_sh, gsems, ssems):
        c = lax.axis_index("c")
        s = lax.axis_index("s")
        w = c * NS + s
        _stage_indices(ei_hbm, w, sidx_v, didx_v)
        tr = N // NS
        pltpu.sync_copy(table_hbm.at[pl.ds(s * tr, tr)],
                        table_sh.at[pl.ds(s * tr, tr)])
        pltpu.sync_copy(zeros_hbm, acc_sh.at[pl.ds(s * ZR, ZR)])
        plsc.subcore_barrier()

        def gather(chunk, b):
            pltpu.async_copy(table_sh.at[sidx_v.at[pl.ds(chunk * K, K)]],
                             rows[b], gsems[b])

        def gather_wait(chunk, b):
            pltpu.make_async_copy(
                table_sh.at[sidx_v.at[pl.ds(chunk * K, K)]], rows[b],
                gsems[b]).wait()

        def scatter(chunk, b):
            pltpu.async_copy(rows[b],
                             acc_sh.at[didx_v.at[pl.ds(chunk * K, K)]],
                             ssems[b], add=True)

        def scatter_wait(b):
            pltpu.make_async_copy(rows[b],
                                  acc_sh.at[didx_v.at[pl.ds(0, K)]],
                                  ssems[b]).wait()

        for b in range(GA):
            gather(b, b)

        def group(g, carry):
            for u in range(NBUF):
                chunk = g * NBUF + u
                gather_wait(chunk, u)
                scatter(chunk, u)
                bf = (u + GA) % NBUF

                @pl.when(chunk + GA < NCHUNK)
                def _():
                    @pl.when(chunk >= NBUF - GA)
                    def _():
                        scatter_wait(bf)

                    gather(chunk + GA, bf)

            return carry

        lax.fori_loop(0, NCHUNK // NBUF, group, 0)
        for b in range(NBUF):
            scatter_wait(b)
        plsc.subcore_barrier()
        pltpu.sync_copy(acc_sh.at[pl.ds(s * ZR, ZR)],
                        out_hbm.at[c, pl.ds(s * ZR, ZR)])

    return agg_kernel(table, ei, zeros_row)


_BM = 1000


def _tc_matmul1(x, w1p):
    def body(x_ref, w_ref, h_ref):
        h_ref[...] = jnp.dot(x_ref[...], w_ref[...],
                             preferred_element_type=jnp.float32)

    return pl.pallas_call(
        body,
        grid=(N // _BM,),
        in_specs=[
            pl.BlockSpec((_BM, 128), lambda i: (i, 0)),
            pl.BlockSpec((128, 64), lambda i: (0, 0)),
        ],
        out_specs=pl.BlockSpec((_BM, 64), lambda i: (i, 0)),
        out_shape=jax.ShapeDtypeStruct((N, 64), jnp.float32),
    )(x, w1p)


def _tc_mid1(h1, degp):
    def body(h_ref, deg_ref, hs_ref, dinv_ref):
        deg = 1.0 + deg_ref[0] + deg_ref[1]
        dinv = lax.rsqrt(deg)
        hs_ref[...] = h_ref[...] * dinv[:, 0:1]
        dinv_ref[...] = dinv

    return pl.pallas_call(
        body,
        grid=(N // _BM,),
        in_specs=[
            pl.BlockSpec((_BM, 64), lambda i: (i, 0)),
            pl.BlockSpec((NC, _BM, 16), lambda i: (0, i, 0)),
        ],
        out_specs=[
            pl.BlockSpec((_BM, 64), lambda i: (i, 0)),
            pl.BlockSpec((_BM, 16), lambda i: (i, 0)),
        ],
        out_shape=[
            jax.ShapeDtypeStruct((N, 64), jnp.float32),
            jax.ShapeDtypeStruct((N, 16), jnp.float32),
        ],
    )(h1, degp)


def _tc_mid2(p, hs1, dinv, b1p, w2p):
    def body(p_ref, hs_ref, dinv_ref, b1_ref, w2_ref, out_ref):
        d1 = dinv_ref[...][:, 0:1]
        a = (p_ref[0] + p_ref[1] + hs_ref[...]) * d1 + b1_ref[...]
        r = jnp.maximum(a, 0.0)
        h2 = jnp.dot(r, w2_ref[...], preferred_element_type=jnp.float32)
        out_ref[...] = h2 * d1

    return pl.pallas_call(
        body,
        grid=(N // _BM,),
        in_specs=[
            pl.BlockSpec((NC, _BM, 64), lambda i: (0, i, 0)),
            pl.BlockSpec((_BM, 64), lambda i: (i, 0)),
            pl.BlockSpec((_BM, 16), lambda i: (i, 0)),
            pl.BlockSpec((1, 64), lambda i: (0, 0)),
            pl.BlockSpec((64, 16), lambda i: (0, 0)),
        ],
        out_specs=pl.BlockSpec((_BM, 16), lambda i: (i, 0)),
        out_shape=jax.ShapeDtypeStruct((N, 16), jnp.float32),
    )(p, hs1, dinv, b1p, w2p)


def _tc_final(q, hs2, dinv, b2p):
    def body(q_ref, hs_ref, dinv_ref, b2_ref, out_ref):
        d1 = dinv_ref[...][:, 0:1]
        z = (q_ref[0] + q_ref[1] + hs_ref[...]) * d1
        logits = z[:, 0:2] + b2_ref[...]
        m = jnp.max(logits, axis=1, keepdims=True)
        e = jnp.exp(logits - m)
        lse = m + jnp.log(e[:, 0:1] + e[:, 1:2])
        out_ref[...] = logits - lse

    return pl.pallas_call(
        body,
        grid=(N // _BM,),
        in_specs=[
            pl.BlockSpec((NC, _BM, 16), lambda i: (0, i, 0)),
            pl.BlockSpec((_BM, 16), lambda i: (i, 0)),
            pl.BlockSpec((_BM, 16), lambda i: (i, 0)),
            pl.BlockSpec((1, 2), lambda i: (0, 0)),
        ],
        out_specs=pl.BlockSpec((_BM, 2), lambda i: (i, 0)),
        out_shape=jax.ShapeDtypeStruct((N, 2), jnp.float32),
    )(q, hs2, dinv, b2p)


def kernel(x, edge_index, W1, b1, W2, b2):
    f_in, h_dim = W1.shape
    c_dim = W2.shape[1]
    w1p = jnp.zeros((f_in, 64), jnp.float32).at[:, :h_dim].set(W1)
    w2p = jnp.zeros((64, 16), jnp.float32).at[:h_dim, :c_dim].set(W2)
    b1p = jnp.zeros((1, 64), jnp.float32).at[0, :h_dim].set(b1)
    b2p = b2.reshape(1, c_dim)
    ones_k = jnp.ones((K, 16), jnp.float32)
    zeros16 = jnp.zeros((ZR, 16), jnp.float32)
    zeros64 = jnp.zeros((ZR, 64), jnp.float32)

    h1 = _tc_matmul1(x, w1p)
    degp = _sc_degree(edge_index, ones_k, zeros16)
    hs1, dinv = _tc_mid1(h1, degp)
    p = _sc_aggregate(hs1, edge_index, zeros64, 64)
    hs2 = _tc_mid2(p, hs1, dinv, b1p, w2p)
    q = _sc_aggregate(hs2, edge_index, zeros16, 16)
    return _tc_final(q, hs2, dinv, b2p)


# speedup vs baseline: 1.7230x; 1.0189x over previous
import functools

import jax
import jax.numpy as jnp
from jax import lax
from jax.experimental import pallas as pl
from jax.experimental.pallas import tpu as pltpu
from jax.experimental.pallas import tpu_sc as plsc

N = 10000
E = 320000
NC = 2
NS = 16
NW = NC * NS
EPW = E // NW
K = 128
NCHUNK = 80
KCAP = NCHUNK * K
PADROW = N
NP = 10240
ZR = NP // NS
NBUF = 2
GA = 1
DEG_Q = 8

_MESH = dict(core_axis_name="c", subcore_axis_name="s")
_SC_PARAMS = pltpu.CompilerParams(use_tc_tiling_on_sc=False)


def _stage_indices(ei_hbm, w, sidx_v, didx_v):
    base = w * EPW
    pltpu.sync_copy(ei_hbm.at[0, pl.ds(base, EPW)], sidx_v.at[pl.ds(0, EPW)])
    pltpu.sync_copy(ei_hbm.at[1, pl.ds(base, EPW)], didx_v.at[pl.ds(0, EPW)])
    zid = jnp.zeros((16,), jnp.int32)
    pad = jnp.full((16,), PADROW, jnp.int32)
    for j in range((KCAP - EPW) // 16):
        sidx_v[pl.ds(EPW + j * 16, 16)] = zid
        didx_v[pl.ds(EPW + j * 16, 16)] = pad


def _sc_degree(ei, ones_k, zeros_row):
    @functools.partial(
        pl.kernel,
        out_type=jax.ShapeDtypeStruct((NC, NP, 16), jnp.float32),
        mesh=plsc.VectorSubcoreMesh(**_MESH),
        scratch_types=[
            pltpu.VMEM((KCAP,), jnp.int32),
            pltpu.VMEM((KCAP,), jnp.int32),
            pltpu.VMEM((K, 16), jnp.float32),
            pltpu.VMEM_SHARED((NP, 16), jnp.float32),
            pltpu.SemaphoreType.DMA,
        ],
        compiler_params=_SC_PARAMS,
    )
    def deg_kernel(ei_hbm, ones_hbm, zeros_hbm, out_hbm, sidx_v, didx_v,
                   ones_v, acc_sh, sem):
        c = lax.axis_index("c")
        s = lax.axis_index("s")
        w = c * NS + s
        _stage_indices(ei_hbm, w, sidx_v, didx_v)
        pltpu.sync_copy(ones_hbm, ones_v)
        pltpu.sync_copy(zeros_hbm, acc_sh.at[pl.ds(s * ZR, ZR)])
        plsc.subcore_barrier()

        def group(g, carry):
            for b in range(DEG_Q):
                pltpu.async_copy(
                    ones_v,
                    acc_sh.at[didx_v.at[pl.ds((g * DEG_Q + b) * K, K)]],
                    sem, add=True)
            for b in range(DEG_Q):
                pltpu.make_async_copy(
                    ones_v,
                    acc_sh.at[didx_v.at[pl.ds((g * DEG_Q + b) * K, K)]],
                    sem).wait()
            return carry

        lax.fori_loop(0, NCHUNK // DEG_Q, group, 0)
        plsc.subcore_barrier()
        pltpu.sync_copy(acc_sh.at[pl.ds(s * ZR, ZR)],
                        out_hbm.at[c, pl.ds(s * ZR, ZR)])

    return deg_kernel(ei, ones_k, zeros_row)


def _sc_aggregate(table, ei, zeros_row, width):
    @functools.partial(
        pl.kernel,
        out_type=jax.ShapeDtypeStruct((NC, NP, width), jnp.float32),
        mesh=plsc.VectorSubcoreMesh(**_MESH),
        scratch_types=[
            pltpu.VMEM((KCAP,), jnp.int32),
            pltpu.VMEM((KCAP,), jnp.int32),
            [pltpu.VMEM((K, width), jnp.float32)] * NBUF,
            pltpu.VMEM_SHARED((N, width), jnp.float32),
            pltpu.VMEM_SHARED((NP, width), jnp.float32),
            [pltpu.SemaphoreType.DMA] * NBUF,
            [pltpu.SemaphoreType.DMA] * NBUF,
        ],
        compiler_params=_SC_PARAMS,
    )
    def agg_kernel(table_hbm, ei_hbm, zeros_hbm, out_hbm, sidx_v, didx_v,
                   rows, table_sh, acc_sh, gsems, ssems):
        c = lax.axis_index("c")
        s = lax.axis_index("s")
        w = c * NS + s
        _stage_indices(ei_hbm, w, sidx_v, didx_v)
        tr = N // NS
        pltpu.sync_copy(table_hbm.at[pl.ds(s * tr, tr)],
                        table_sh.at[pl.ds(s * tr, tr)])
        pltpu.sync_copy(zeros_hbm, acc_sh.at[pl.ds(s * ZR, ZR)])
        plsc.subcore_barrier()

        def gather(chunk, b):
            pltpu.async_copy(table_sh.at[sidx_v.at[pl.ds(chunk * K, K)]],
                             rows[b], gsems[b])

        def gather_wait(chunk, b):
            pltpu.make_async_copy(
                table_sh.at[sidx_v.at[pl.ds(chunk * K, K)]], rows[b],
                gsems[b]).wait()

        def scatter(chunk, b):
            pltpu.async_copy(rows[b],
                             acc_sh.at[didx_v.at[pl.ds(chunk * K, K)]],
                             ssems[b], add=True)

        def scatter_wait(b):
            pltpu.make_async_copy(rows[b],
                                  acc_sh.at[didx_v.at[pl.ds(0, K)]],
                                  ssems[b]).wait()

        for b in range(GA):
            gather(b, b)

        def group(g, carry):
            for u in range(NBUF):
                chunk = g * NBUF + u
                gather_wait(chunk, u)
                scatter(chunk, u)
                bf = (u + GA) % NBUF

                @pl.when(chunk + GA < NCHUNK)
                def _():
                    @pl.when(chunk >= NBUF - GA)
                    def _():
                        scatter_wait(bf)

                    gather(chunk + GA, bf)

            return carry

        lax.fori_loop(0, NCHUNK // NBUF, group, 0)
        for b in range(NBUF):
            scatter_wait(b)
        plsc.subcore_barrier()
        pltpu.sync_copy(acc_sh.at[pl.ds(s * ZR, ZR)],
                        out_hbm.at[c, pl.ds(s * ZR, ZR)])

    return agg_kernel(table, ei, zeros_row)


_BM = 2000


def _tc_matmul1(x, w1p):
    def body(x_ref, w_ref, h_ref):
        h_ref[...] = jnp.dot(x_ref[...], w_ref[...],
                             preferred_element_type=jnp.float32)

    return pl.pallas_call(
        body,
        grid=(N // _BM,),
        in_specs=[
            pl.BlockSpec((_BM, 128), lambda i: (i, 0)),
            pl.BlockSpec((128, 64), lambda i: (0, 0)),
        ],
        out_specs=pl.BlockSpec((_BM, 64), lambda i: (i, 0)),
        out_shape=jax.ShapeDtypeStruct((N, 64), jnp.float32),
    )(x, w1p)


def _tc_mid1(h1, degp):
    def body(h_ref, deg_ref, hs_ref, dinv_ref):
        deg = 1.0 + deg_ref[0] + deg_ref[1]
        dinv = lax.rsqrt(deg)
        hs_ref[...] = h_ref[...] * dinv[:, 0:1]
        dinv_ref[...] = dinv

    return pl.pallas_call(
        body,
        grid=(N // _BM,),
        in_specs=[
            pl.BlockSpec((_BM, 64), lambda i: (i, 0)),
            pl.BlockSpec((NC, _BM, 16), lambda i: (0, i, 0)),
        ],
        out_specs=[
            pl.BlockSpec((_BM, 64), lambda i: (i, 0)),
            pl.BlockSpec((_BM, 16), lambda i: (i, 0)),
        ],
        out_shape=[
            jax.ShapeDtypeStruct((N, 64), jnp.float32),
            jax.ShapeDtypeStruct((N, 16), jnp.float32),
        ],
    )(h1, degp)


def _tc_mid2(p, hs1, dinv, b1p, w2p):
    def body(p_ref, hs_ref, dinv_ref, b1_ref, w2_ref, out_ref):
        d1 = dinv_ref[...][:, 0:1]
        a = (p_ref[0] + p_ref[1] + hs_ref[...]) * d1 + b1_ref[...]
        r = jnp.maximum(a, 0.0)
        h2 = jnp.dot(r, w2_ref[...], preferred_element_type=jnp.float32)
        out_ref[...] = h2 * d1

    return pl.pallas_call(
        body,
        grid=(N // _BM,),
        in_specs=[
            pl.BlockSpec((NC, _BM, 64), lambda i: (0, i, 0)),
            pl.BlockSpec((_BM, 64), lambda i: (i, 0)),
            pl.BlockSpec((_BM, 16), lambda i: (i, 0)),
            pl.BlockSpec((1, 64), lambda i: (0, 0)),
            pl.BlockSpec((64, 16), lambda i: (0, 0)),
        ],
        out_specs=pl.BlockSpec((_BM, 16), lambda i: (i, 0)),
        out_shape=jax.ShapeDtypeStruct((N, 16), jnp.float32),
    )(p, hs1, dinv, b1p, w2p)


def _tc_final(q, hs2, dinv, b2p):
    def body(q_ref, hs_ref, dinv_ref, b2_ref, out_ref):
        d1 = dinv_ref[...][:, 0:1]
        z = (q_ref[0] + q_ref[1] + hs_ref[...]) * d1
        logits = z + b2_ref[...]
        l0 = logits[:, 0:1]
        l1 = logits[:, 1:2]
        m = jnp.maximum(l0, l1)
        lse = m + jnp.log(jnp.exp(l0 - m) + jnp.exp(l1 - m))
        out_ref[...] = logits - lse

    return pl.pallas_call(
        body,
        grid=(N // _BM,),
        in_specs=[
            pl.BlockSpec((NC, _BM, 16), lambda i: (0, i, 0)),
            pl.BlockSpec((_BM, 16), lambda i: (i, 0)),
            pl.BlockSpec((_BM, 16), lambda i: (i, 0)),
            pl.BlockSpec((1, 16), lambda i: (0, 0)),
        ],
        out_specs=pl.BlockSpec((_BM, 16), lambda i: (i, 0)),
        out_shape=jax.ShapeDtypeStruct((N, 16), jnp.float32),
    )(q, hs2, dinv, b2p)


def kernel(x, edge_index, W1, b1, W2, b2):
    f_in, h_dim = W1.shape
    c_dim = W2.shape[1]
    w1p = jnp.zeros((f_in, 64), jnp.float32).at[:, :h_dim].set(W1)
    w2p = jnp.zeros((64, 16), jnp.float32).at[:h_dim, :c_dim].set(W2)
    b1p = jnp.zeros((1, 64), jnp.float32).at[0, :h_dim].set(b1)
    b2p = jnp.zeros((1, 16), jnp.float32).at[0, :c_dim].set(b2)
    ones_k = jnp.ones((K, 16), jnp.float32)
    zeros16 = jnp.zeros((ZR, 16), jnp.float32)
    zeros64 = jnp.zeros((ZR, 64), jnp.float32)

    h1 = _tc_matmul1(x, w1p)
    degp = _sc_degree(edge_index, ones_k, zeros16)
    hs1, dinv = _tc_mid1(h1, degp)
    p = _sc_aggregate(hs1, edge_index, zeros64, 64)
    hs2 = _tc_mid2(p, hs1, dinv, b1p, w2p)
    q = _sc_aggregate(hs2, edge_index, zeros16, 16)
    return _tc_final(q, hs2, dinv, b2p)[:, :c_dim]
